# baseline (speedup 1.0000x reference)
import functools

import jax
import jax.numpy as jnp
from jax import lax
from jax.experimental import pallas as pl
from jax.experimental.pallas import tpu as pltpu
from jax.experimental.pallas import tpu_sc as plsc

NUM_NODES = 1_000_000
BATCH = 16384
DIM = 64


@functools.cache
def _make_gather():
    info = plsc.get_sparse_core_info()
    nc, ns = info.num_cores, info.num_subcores
    nw = nc * ns
    b_per_w = BATCH // nw

    mesh = plsc.VectorSubcoreMesh(core_axis_name="c", subcore_axis_name="s")

    @functools.partial(
        pl.kernel,
        mesh=mesh,
        out_type=jax.ShapeDtypeStruct((nw, DIM * b_per_w), jnp.float32),
        scratch_types=[
            pltpu.VMEM((b_per_w,), jnp.int32),
            pltpu.VMEM((DIM * b_per_w,), jnp.float32),
            pltpu.SemaphoreType.DMA,
        ],
        compiler_params=pltpu.CompilerParams(use_tc_tiling_on_sc=False),
    )
    def gather_kernel(table_t_hbm, idx_hbm, out_hbm, idx_v, st1, sem):
        wid = lax.axis_index("s") * nc + lax.axis_index("c")
        base = wid * b_per_w
        pltpu.sync_copy(idx_hbm.at[pl.ds(base, b_per_w)], idx_v)
        for e in range(DIM):
            pltpu.make_async_copy(
                table_t_hbm.at[e].at[idx_v],
                st1.at[pl.ds(e * b_per_w, b_per_w)],
                sem,
            ).start()
        pltpu.make_async_copy(
            table_t_hbm.at[0].at[pl.ds(0, DIM * b_per_w)], st1, sem
        ).wait()
        pltpu.sync_copy(st1, out_hbm.at[wid])

    return gather_kernel, nw, b_per_w


@jax.jit
def kernel(index, lookup_table):
    gather, nw, b_per_w = _make_gather()
    out3 = gather(lookup_table.T, index.astype(jnp.int32))
    out4 = out3.reshape(nw, DIM, b_per_w)
    return out4.transpose(0, 2, 1).reshape(BATCH, DIM)


# speedup vs baseline: 23.2095x; 23.2095x over previous
import functools

import jax
import jax.numpy as jnp
from jax import lax
from jax.experimental import pallas as pl
from jax.experimental.pallas import tpu as pltpu
from jax.experimental.pallas import tpu_sc as plsc

NUM_NODES = 1_000_000
BATCH = 16384
DIM = 64

PAD_NODES = 1_000_064
TILES_PER_ROW = PAD_NODES // 128
ROW_STRIDE = TILES_PER_ROW * 1024
FLAT = 8 * ROW_STRIDE
GATHER_SPAN = 7_999_616


@functools.cache
def _make_gather():
    info = plsc.get_sparse_core_info()
    nc, ns = info.num_cores, info.num_subcores
    nw = nc * ns
    b_per_w = BATCH // nw

    mesh = plsc.VectorSubcoreMesh(core_axis_name="c", subcore_axis_name="s")

    @functools.partial(
        pl.kernel,
        mesh=mesh,
        out_type=jax.ShapeDtypeStruct((nw, DIM * b_per_w), jnp.float32),
        scratch_types=[
            pltpu.VMEM((b_per_w,), jnp.int32),
            pltpu.VMEM((b_per_w,), jnp.int32),
            pltpu.VMEM((DIM * b_per_w,), jnp.float32),
            pltpu.SemaphoreType.DMA,
        ],
        compiler_params=pltpu.CompilerParams(use_tc_tiling_on_sc=False),
    )
    def gather_kernel(flat_hbm, idx_hbm, out_hbm, idx_v, off_v, st1, sem):
        wid = lax.axis_index("s") * nc + lax.axis_index("c")
        base = wid * b_per_w
        pltpu.sync_copy(idx_hbm.at[pl.ds(base, b_per_w)], idx_v)
        for i in range(b_per_w // 16):
            v = idx_v[pl.ds(i * 16, 16)]
            off_v[pl.ds(i * 16, 16)] = v + (v >> 7) * 896
        for e in range(DIM):
            c_e = (e // 8) * ROW_STRIDE + (e % 8) * 128
            pltpu.make_async_copy(
                flat_hbm.at[pl.ds(c_e, GATHER_SPAN)].at[off_v],
                st1.at[pl.ds(e * b_per_w, b_per_w)],
                sem,
            ).start()
        pltpu.make_async_copy(
            flat_hbm.at[pl.ds(0, DIM * b_per_w)], st1, sem
        ).wait()
        pltpu.sync_copy(st1, out_hbm.at[wid])

    return gather_kernel, nw, b_per_w


@jax.jit
def kernel(index, lookup_table):
    gather, nw, b_per_w = _make_gather()
    padded = jnp.pad(lookup_table, ((0, PAD_NODES - NUM_NODES), (0, 0)))
    flat = (
        padded.T.reshape(8, 8, TILES_PER_ROW, 128)
        .transpose(0, 2, 1, 3)
        .reshape(FLAT)
    )
    out3 = gather(flat, index.astype(jnp.int32))
    out4 = out3.reshape(nw, DIM, b_per_w)
    return out4.transpose(0, 2, 1).reshape(BATCH, DIM)


# speedup vs baseline: 23.7866x; 1.0249x over previous
import functools

import jax
import jax.numpy as jnp
from jax import lax
from jax.experimental import pallas as pl
from jax.experimental.pallas import tpu as pltpu
from jax.experimental.pallas import tpu_sc as plsc

NUM_NODES = 1_000_000
BATCH = 16384
DIM = 64

PAD_NODES = 1_000_064
TILES_PER_ROW = PAD_NODES // 128
ROW_STRIDE = TILES_PER_ROW * 1024
FLAT = 8 * ROW_STRIDE
GATHER_SPAN = 7_999_616


@functools.cache
def _make_gather():
    info = plsc.get_sparse_core_info()
    nc, ns = info.num_cores, info.num_subcores
    nw = nc * ns
    b_per_w = BATCH // nw

    mesh = plsc.VectorSubcoreMesh(core_axis_name="c", subcore_axis_name="s")

    lane_blocks = b_per_w // 128

    @functools.partial(
        pl.kernel,
        mesh=mesh,
        out_type=jax.ShapeDtypeStruct((8, BATCH // 128, 8, 128), jnp.float32),
        scratch_types=[
            pltpu.VMEM((b_per_w,), jnp.int32),
            pltpu.VMEM((b_per_w,), jnp.int32),
            pltpu.VMEM((DIM, b_per_w), jnp.float32),
            pltpu.SemaphoreType.DMA,
            pltpu.SemaphoreType.DMA,
        ],
        compiler_params=pltpu.CompilerParams(use_tc_tiling_on_sc=False),
    )
    def gather_kernel(flat_hbm, idx_hbm, out_hbm, idx_v, off_v, st1, sem, osem):
        wid = lax.axis_index("s") * nc + lax.axis_index("c")
        base = wid * b_per_w
        pltpu.sync_copy(idx_hbm.at[pl.ds(base, b_per_w)], idx_v)
        for i in range(b_per_w // 16):
            v = idx_v[pl.ds(i * 16, 16)]
            off_v[pl.ds(i * 16, 16)] = v + (v >> 7) * 896
        for e in range(DIM):
            c_e = (e // 8) * ROW_STRIDE + (e % 8) * 128
            pltpu.make_async_copy(
                flat_hbm.at[pl.ds(c_e, GATHER_SPAN)].at[off_v],
                st1.at[e],
                sem,
            ).start()
        pltpu.make_async_copy(
            flat_hbm.at[pl.ds(0, DIM * b_per_w)], st1, sem
        ).wait()
        for r in range(8):
            for j in range(lane_blocks):
                pltpu.make_async_copy(
                    st1.at[pl.ds(8 * r, 8), pl.ds(128 * j, 128)],
                    out_hbm.at[r, wid * lane_blocks + j],
                    osem,
                ).start()
        pltpu.make_async_copy(
            flat_hbm.at[pl.ds(0, DIM * b_per_w)], st1, osem
        ).wait()

    return gather_kernel, nw, b_per_w


@jax.jit
def kernel(index, lookup_table):
    gather, nw, b_per_w = _make_gather()
    padded = jnp.pad(lookup_table, ((0, PAD_NODES - NUM_NODES), (0, 0)))
    flat = (
        padded.T.reshape(8, 8, TILES_PER_ROW, 128)
        .transpose(0, 2, 1, 3)
        .reshape(FLAT)
    )
    out4 = gather(flat, index.astype(jnp.int32))
    return out4.transpose(1, 3, 0, 2).reshape(BATCH, DIM)
